# Initial kernel scaffold; baseline (speedup 1.0000x reference)
import jax
import jax.numpy as jnp
from jax.experimental import pallas as pl


def _sincos_body(p_ref, c_ref, s_ref):
    p = p_ref[...]
    c_ref[...] = jnp.cos(p)
    s_ref[...] = jnp.sin(p)


def kernel(x, edges, phases):
    E = phases.shape[0]
    pm = phases.reshape(6250, 1024)
    c, s = pl.pallas_call(
        _sincos_body,
        grid=(125,),
        in_specs=[pl.BlockSpec((50, 1024), lambda i: (i, 0))],
        out_specs=[pl.BlockSpec((50, 1024), lambda i: (i, 0))] * 2,
        out_shape=[jax.ShapeDtypeStruct((6250, 1024), jnp.float32)] * 2,
    )(pm)
    c = c.reshape(E)
    s = s.reshape(E)
    u = edges[:, 0]
    v = edges[:, 1]
    hu = x[u]
    hv = x[v]
    tu = jnp.stack([hu[:, 0] * c - hu[:, 1] * s, hu[:, 0] * s + hu[:, 1] * c], axis=1)
    tv = jnp.stack([hv[:, 0] * c + hv[:, 1] * s, -hv[:, 0] * s + hv[:, 1] * c], axis=1)
    out = jnp.zeros_like(x).at[v].add(tu).at[u].add(tv)
    return x + out


# speedup vs baseline: 1.0742x; 1.0742x over previous
import jax
import jax.numpy as jnp
from jax.experimental import pallas as pl


def _sincos_body(p_ref, c_ref, s_ref):
    p = p_ref[...]
    c_ref[...] = jnp.cos(p)
    s_ref[...] = jnp.sin(p)


def kernel(x, edges, phases):
    E = phases.shape[0]
    pm = phases.reshape(50000, 128)
    c, s = pl.pallas_call(
        _sincos_body,
        grid=(50,),
        in_specs=[pl.BlockSpec((1000, 128), lambda i: (i, 0))],
        out_specs=[pl.BlockSpec((1000, 128), lambda i: (i, 0))] * 2,
        out_shape=[jax.ShapeDtypeStruct((50000, 128), jnp.float32)] * 2,
    )(pm)
    c = c.reshape(E)
    s = s.reshape(E)
    u = edges[:, 0]
    v = edges[:, 1]
    hu = x[u]
    hv = x[v]
    tu = jnp.stack([hu[:, 0] * c - hu[:, 1] * s, hu[:, 0] * s + hu[:, 1] * c], axis=1)
    tv = jnp.stack([hv[:, 0] * c + hv[:, 1] * s, -hv[:, 0] * s + hv[:, 1] * c], axis=1)
    out = jnp.zeros_like(x).at[v].add(tu).at[u].add(tv)
    return x + out


# speedup vs baseline: 56.8921x; 52.9606x over previous
import jax
import jax.numpy as jnp
from jax import lax
from jax.experimental import pallas as pl
from jax.experimental.pallas import tpu as pltpu
from jax.experimental.pallas import tpu_sc as plsc

N_NODES = 100000
NPAD = 100096
AWORDS = 2 * NPAD
HALF = NPAD // 2
COLS = 128
PAD_ROWS = 50048
NCORES = 2
NSUB = 16
NTILES = NCORES * NSUB
RPT = PAD_ROWS // NTILES
K = 17
NCHUNK = RPT // K
LANES = 16

_CP = pltpu.CompilerParams(needs_layout_passes=False, use_tc_tiling_on_sc=False)
_MESH = plsc.VectorSubcoreMesh(core_axis_name="c", subcore_axis_name="s",
                               num_cores=NCORES, num_subcores=NSUB)


def _sincos_body(p_ref, cs_ref):
    p = p_ref[...]
    c = jnp.cos(p).astype(jnp.bfloat16)
    s = jnp.sin(p).astype(jnp.bfloat16)
    cb = lax.bitcast_convert_type(c, jnp.uint16).astype(jnp.int32)
    sb = lax.bitcast_convert_type(s, jnp.uint16).astype(jnp.int32)
    cs_ref[...] = (sb << 16) | cb


def _sincos_pack(p2):
    return pl.pallas_call(
        _sincos_body,
        grid=(8,),
        in_specs=[pl.BlockSpec((PAD_ROWS // 8, COLS), lambda i: (i, 0))],
        out_specs=pl.BlockSpec((PAD_ROWS // 8, COLS), lambda i: (i, 0)),
        out_shape=jax.ShapeDtypeStruct((PAD_ROWS, COLS), jnp.int32),
    )(p2)


def _unpack_pair(w):
    a = plsc.bitcast(lax.shift_left(w, 16), jnp.float32)
    b = plsc.bitcast(lax.bitwise_and(w, jnp.int32(-65536)), jnp.float32)
    return a, b


def _rotate_body(xp_hbm, u_hbm, v_hbm, cs_hbm,
                 va_hbm, vb_hbm, vc_hbm, vd_hbm,
                 xp_v, u_in, v_in, cs_in, va_b, vb_b, vc_b, vd_b, sem):
    c = lax.axis_index("c")
    s = lax.axis_index("s")
    wid = c * NSUB + s
    pltpu.sync_copy(xp_hbm, xp_v)
    rbeg = wid * RPT

    def chunk(i, carry):
        r0 = rbeg + i * K
        sl_r = pl.ds(r0, K)
        d1 = pltpu.async_copy(u_hbm.at[sl_r], u_in, sem)
        d2 = pltpu.async_copy(v_hbm.at[sl_r], v_in, sem)
        d3 = pltpu.async_copy(cs_hbm.at[sl_r], cs_in, sem)
        d1.wait()
        d2.wait()
        d3.wait()
        for k in range(K):
            for m in range(COLS // LANES):
                sl = pl.ds(LANES * m, LANES)
                u16 = u_in[k, sl]
                v16 = v_in[k, sl]
                cc, ss = _unpack_pair(cs_in[k, sl])
                xu0, xu1 = _unpack_pair(plsc.load_gather(xp_v, [u16]))
                xv0, xv1 = _unpack_pair(plsc.load_gather(xp_v, [v16]))
                va_b[k, sl] = xu0 * cc - xu1 * ss
                vb_b[k, sl] = xu0 * ss + xu1 * cc
                vc_b[k, sl] = xv0 * cc + xv1 * ss
                vd_b[k, sl] = xv1 * cc - xv0 * ss
        o1 = pltpu.async_copy(va_b, va_hbm.at[sl_r], sem)
        o2 = pltpu.async_copy(vb_b, vb_hbm.at[sl_r], sem)
        o3 = pltpu.async_copy(vc_b, vc_hbm.at[sl_r], sem)
        o4 = pltpu.async_copy(vd_b, vd_hbm.at[sl_r], sem)
        o1.wait()
        o2.wait()
        o3.wait()
        o4.wait()
        return carry

    lax.fori_loop(0, NCHUNK, chunk, 0)


_rotate_call = pl.kernel(
    _rotate_body,
    out_type=[jax.ShapeDtypeStruct((PAD_ROWS, COLS), jnp.float32)] * 4,
    mesh=_MESH,
    compiler_params=_CP,
    scratch_types=[
        pltpu.VMEM((NPAD,), jnp.int32),
        pltpu.VMEM((K, COLS), jnp.int32),
        pltpu.VMEM((K, COLS), jnp.int32),
        pltpu.VMEM((K, COLS), jnp.int32),
        pltpu.VMEM((K, COLS), jnp.float32),
        pltpu.VMEM((K, COLS), jnp.float32),
        pltpu.VMEM((K, COLS), jnp.float32),
        pltpu.VMEM((K, COLS), jnp.float32),
        pltpu.SemaphoreType.DMA,
    ],
)


def _accum_body(u_hbm, v_hbm, va_hbm, vb_hbm, vc_hbm, vd_hbm, z_hbm,
                out_hbm, u_in, v_in, va_b, vb_b, vc_b, vd_b, acc, sem):
    c = lax.axis_index("c")
    s = lax.axis_index("s")
    wid = c * NSUB + s
    rbeg = wid * RPT

    for p in range(2):
        pltpu.sync_copy(z_hbm, acc)
        lo = jnp.int32(p * HALF)
        hi = jnp.int32((p + 1) * HALF)

        def chunk(i, carry):
            r0 = rbeg + i * K
            sl_r = pl.ds(r0, K)
            d1 = pltpu.async_copy(u_hbm.at[sl_r], u_in, sem)
            d2 = pltpu.async_copy(v_hbm.at[sl_r], v_in, sem)
            d3 = pltpu.async_copy(va_hbm.at[sl_r], va_b, sem)
            d4 = pltpu.async_copy(vb_hbm.at[sl_r], vb_b, sem)
            d5 = pltpu.async_copy(vc_hbm.at[sl_r], vc_b, sem)
            d6 = pltpu.async_copy(vd_hbm.at[sl_r], vd_b, sem)
            d1.wait()
            d2.wait()
            d3.wait()
            d4.wait()
            d5.wait()
            d6.wait()
            for k in range(K):
                for m in range(COLS // LANES):
                    sl = pl.ds(LANES * m, LANES)
                    u16 = u_in[k, sl]
                    v16 = v_in[k, sl]
                    mv = jnp.logical_and(v16 >= lo, v16 < hi)
                    mu = jnp.logical_and(u16 >= lo, u16 < hi)
                    bv = jnp.where(mv, lax.shift_left(v16 - lo, 1), 0)
                    bu = jnp.where(mu, lax.shift_left(u16 - lo, 1), 0)
                    plsc.addupdate_scatter(acc, [bv], va_b[k, sl], mask=mv)
                    plsc.addupdate_scatter(acc, [bv + 1], vb_b[k, sl], mask=mv)
                    plsc.addupdate_scatter(acc, [bu], vc_b[k, sl], mask=mu)
                    plsc.addupdate_scatter(acc, [bu + 1], vd_b[k, sl], mask=mu)
            return carry

        lax.fori_loop(0, NCHUNK, chunk, 0)
        pltpu.sync_copy(
            acc, out_hbm.at[pl.ds(wid * AWORDS + p * NPAD, NPAD)])


_accum_call = pl.kernel(
    _accum_body,
    out_type=jax.ShapeDtypeStruct((NTILES * AWORDS,), jnp.float32),
    mesh=_MESH,
    compiler_params=_CP,
    scratch_types=[
        pltpu.VMEM((K, COLS), jnp.int32),
        pltpu.VMEM((K, COLS), jnp.int32),
        pltpu.VMEM((K, COLS), jnp.float32),
        pltpu.VMEM((K, COLS), jnp.float32),
        pltpu.VMEM((K, COLS), jnp.float32),
        pltpu.VMEM((K, COLS), jnp.float32),
        pltpu.VMEM((NPAD,), jnp.float32),
        pltpu.SemaphoreType.DMA,
    ],
)


_FCOLS = 5888


def _finish_body(parts_ref, x_ref, o_ref):
    o_ref[...] = jnp.sum(parts_ref[...], axis=0, keepdims=True) + x_ref[...]


def _finish(parts, xf):
    return pl.pallas_call(
        _finish_body,
        grid=(AWORDS // _FCOLS,),
        in_specs=[pl.BlockSpec((NTILES, _FCOLS), lambda i: (0, i)),
                  pl.BlockSpec((1, _FCOLS), lambda i: (0, i))],
        out_specs=pl.BlockSpec((1, _FCOLS), lambda i: (0, i)),
        out_shape=jax.ShapeDtypeStruct((1, AWORDS), jnp.float32),
    )(parts, xf)


def kernel(x, edges, phases):
    E = phases.shape[0]
    pad = PAD_ROWS * COLS - E
    u = edges[:, 0].astype(jnp.int32)
    v = edges[:, 1].astype(jnp.int32)
    u2 = jnp.concatenate([u, jnp.full((pad,), N_NODES, jnp.int32)]).reshape(
        PAD_ROWS, COLS)
    v2 = jnp.concatenate([v, jnp.full((pad,), N_NODES, jnp.int32)]).reshape(
        PAD_ROWS, COLS)
    p2 = jnp.concatenate([phases, jnp.zeros((pad,), jnp.float32)]).reshape(
        PAD_ROWS, COLS)
    cs = _sincos_pack(p2)
    xb0 = lax.bitcast_convert_type(x[:, 0].astype(jnp.bfloat16),
                                   jnp.uint16).astype(jnp.int32)
    xb1 = lax.bitcast_convert_type(x[:, 1].astype(jnp.bfloat16),
                                   jnp.uint16).astype(jnp.int32)
    xp = jnp.concatenate([(xb1 << 16) | xb0,
                          jnp.zeros((NPAD - N_NODES,), jnp.int32)])
    va, vb, vc, vd = _rotate_call(xp, u2, v2, cs)
    zeros = jnp.zeros((NPAD,), jnp.float32)
    parts = _accum_call(u2, v2, va, vb, vc, vd, zeros)
    xf = jnp.concatenate([x.reshape(-1),
                          jnp.zeros((AWORDS - 2 * N_NODES,), jnp.float32)])
    total = _finish(parts.reshape(NTILES, AWORDS), xf.reshape(1, AWORDS))
    return total.reshape(-1)[:2 * N_NODES].reshape(N_NODES, 2)


# speedup vs baseline: 96.1092x; 1.6893x over previous
import jax
import jax.numpy as jnp
from jax import lax
from jax.experimental import pallas as pl
from jax.experimental.pallas import tpu as pltpu
from jax.experimental.pallas import tpu_sc as plsc

N_NODES = 100000
NPAD = 100096
AWORDS = 2 * NPAD
COLS = 128
PAD_ROWS = 50048
NCORES = 2
NSUB = 16
NTILES = NCORES * NSUB
RPT = PAD_ROWS // NTILES
K = 17
NCHUNK = RPT // K
LANES = 16

_CP = pltpu.CompilerParams(needs_layout_passes=False, use_tc_tiling_on_sc=False)
_MESH = plsc.VectorSubcoreMesh(core_axis_name="c", subcore_axis_name="s",
                               num_cores=NCORES, num_subcores=NSUB)


def _sincos_body(p_ref, cs_ref):
    p = p_ref[...]
    c = jnp.cos(p).astype(jnp.bfloat16)
    s = jnp.sin(p).astype(jnp.bfloat16)
    cb = lax.bitcast_convert_type(c, jnp.uint16).astype(jnp.int32)
    sb = lax.bitcast_convert_type(s, jnp.uint16).astype(jnp.int32)
    cs_ref[...] = (sb << 16) | cb


def _sincos_pack(p2):
    return pl.pallas_call(
        _sincos_body,
        grid=(8,),
        in_specs=[pl.BlockSpec((PAD_ROWS // 8, COLS), lambda i: (i, 0))],
        out_specs=pl.BlockSpec((PAD_ROWS // 8, COLS), lambda i: (i, 0)),
        out_shape=jax.ShapeDtypeStruct((PAD_ROWS, COLS), jnp.int32),
    )(p2)


def _unpack_pair(w):
    a = plsc.bitcast(lax.shift_left(w, 16), jnp.float32)
    b = plsc.bitcast(lax.bitwise_and(w, jnp.int32(-65536)), jnp.float32)
    return a, b


def _rotate_body(xp_hbm, u_hbm, v_hbm, cs_hbm,
                 va_hbm, vb_hbm, vc_hbm, vd_hbm,
                 xp_v, u0, v0, cs0, u1, v1, cs1,
                 va_b, vb_b, vc_b, vd_b, si0, si1, so):
    c = lax.axis_index("c")
    s = lax.axis_index("s")
    wid = c * NSUB + s
    pltpu.sync_copy(xp_hbm, xp_v)
    rbeg = wid * RPT
    ins = ((u0, v0, cs0, si0), (u1, v1, cs1, si1))
    srcs = (u_hbm, v_hbm, cs_hbm)
    outs_hbm = (va_hbm, vb_hbm, vc_hbm, vd_hbm)
    outs_b = (va_b, vb_b, vc_b, vd_b)

    def fire(i, b):
        sl_r = pl.ds(rbeg + i * K, K)
        bu, bv, bc, sem = ins[b]
        pltpu.async_copy(u_hbm.at[sl_r], bu, sem)
        pltpu.async_copy(v_hbm.at[sl_r], bv, sem)
        pltpu.async_copy(cs_hbm.at[sl_r], bc, sem)

    def drain_in(b):
        bu, bv, bc, sem = ins[b]
        for src, dst in zip(srcs, (bu, bv, bc)):
            pltpu.make_async_copy(src.at[pl.ds(0, K)], dst, sem).wait()

    def drain_out():
        for src, dst in zip(outs_b, outs_hbm):
            pltpu.make_async_copy(src, dst.at[pl.ds(0, K)], so).wait()

    fire(0, 0)

    def group(g, carry):
        for b in range(2):
            i = g * 2 + b
            drain_in(b)

            @pl.when(i + 1 < NCHUNK)
            def _():
                fire(i + 1, 1 - b)

            @pl.when(i >= 1)
            def _():
                drain_out()

            bu, bv, bc, _ = ins[b]
            for k in range(K):
                for m in range(COLS // LANES):
                    sl = pl.ds(LANES * m, LANES)
                    u16 = bu[k, sl]
                    v16 = bv[k, sl]
                    cc, ss = _unpack_pair(bc[k, sl])
                    xu0, xu1 = _unpack_pair(plsc.load_gather(xp_v, [u16]))
                    xv0, xv1 = _unpack_pair(plsc.load_gather(xp_v, [v16]))
                    va_b[k, sl] = xu0 * cc - xu1 * ss
                    vb_b[k, sl] = xu0 * ss + xu1 * cc
                    vc_b[k, sl] = xv0 * cc + xv1 * ss
                    vd_b[k, sl] = xv1 * cc - xv0 * ss
            sl_r = pl.ds(rbeg + i * K, K)
            for src, dst in zip(outs_b, outs_hbm):
                pltpu.async_copy(src, dst.at[sl_r], so)
        return carry

    lax.fori_loop(0, NCHUNK // 2, group, 0)
    drain_out()


_rotate_call = pl.kernel(
    _rotate_body,
    out_type=[jax.ShapeDtypeStruct((PAD_ROWS, COLS), jnp.float32)] * 4,
    mesh=_MESH,
    compiler_params=_CP,
    scratch_types=[
        pltpu.VMEM((NPAD,), jnp.int32),
        pltpu.VMEM((K, COLS), jnp.int32),
        pltpu.VMEM((K, COLS), jnp.int32),
        pltpu.VMEM((K, COLS), jnp.int32),
        pltpu.VMEM((K, COLS), jnp.int32),
        pltpu.VMEM((K, COLS), jnp.int32),
        pltpu.VMEM((K, COLS), jnp.int32),
        pltpu.VMEM((K, COLS), jnp.float32),
        pltpu.VMEM((K, COLS), jnp.float32),
        pltpu.VMEM((K, COLS), jnp.float32),
        pltpu.VMEM((K, COLS), jnp.float32),
        pltpu.SemaphoreType.DMA,
        pltpu.SemaphoreType.DMA,
        pltpu.SemaphoreType.DMA,
    ],
)


def _accum_body(u_hbm, v_hbm, va_hbm, vb_hbm, vc_hbm, vd_hbm, z_hbm,
                out_hbm, u0, v0, a0, c0, u1, v1, a1, c1, acc, s0, s1):
    c = lax.axis_index("c")
    s = lax.axis_index("s")
    wid = c * NSUB + s
    rbeg = wid * RPT
    ins = ((u0, v0, a0, c0, s0), (u1, v1, a1, c1, s1))

    for p in range(2):
        A_hbm = va_hbm if p == 0 else vb_hbm
        C_hbm = vc_hbm if p == 0 else vd_hbm
        srcs = (u_hbm, v_hbm, A_hbm, C_hbm)
        pltpu.sync_copy(z_hbm, acc)

        def fire(i, b):
            sl_r = pl.ds(rbeg + i * K, K)
            bufs = ins[b]
            for src, dst in zip(srcs, bufs[:4]):
                pltpu.async_copy(src.at[sl_r], dst, bufs[4])

        def drain(b):
            bufs = ins[b]
            for src, dst in zip(srcs, bufs[:4]):
                pltpu.make_async_copy(src.at[pl.ds(0, K)], dst, bufs[4]).wait()

        fire(0, 0)

        def group(g, carry):
            for b in range(2):
                i = g * 2 + b
                drain(b)

                @pl.when(i + 1 < NCHUNK)
                def _():
                    fire(i + 1, 1 - b)

                bu, bv, ba, bcv, _ = ins[b]
                for k in range(K):
                    for m in range(COLS // LANES):
                        sl = pl.ds(LANES * m, LANES)
                        plsc.addupdate_scatter(acc, [bv[k, sl]], ba[k, sl])
                        plsc.addupdate_scatter(acc, [bu[k, sl]], bcv[k, sl])
            return carry

        lax.fori_loop(0, NCHUNK // 2, group, 0)
        pltpu.sync_copy(acc, out_hbm.at[pl.ds(wid * AWORDS + p * NPAD, NPAD)])


_accum_call = pl.kernel(
    _accum_body,
    out_type=jax.ShapeDtypeStruct((NTILES * AWORDS,), jnp.float32),
    mesh=_MESH,
    compiler_params=_CP,
    scratch_types=[
        pltpu.VMEM((K, COLS), jnp.int32),
        pltpu.VMEM((K, COLS), jnp.int32),
        pltpu.VMEM((K, COLS), jnp.float32),
        pltpu.VMEM((K, COLS), jnp.float32),
        pltpu.VMEM((K, COLS), jnp.int32),
        pltpu.VMEM((K, COLS), jnp.int32),
        pltpu.VMEM((K, COLS), jnp.float32),
        pltpu.VMEM((K, COLS), jnp.float32),
        pltpu.VMEM((NPAD,), jnp.float32),
        pltpu.SemaphoreType.DMA,
        pltpu.SemaphoreType.DMA,
    ],
)


_FCOLS = 5888


def _finish_body(parts_ref, x_ref, o_ref):
    o_ref[...] = jnp.sum(parts_ref[...], axis=0, keepdims=True) + x_ref[...]


def _finish(parts, xf):
    return pl.pallas_call(
        _finish_body,
        grid=(AWORDS // _FCOLS,),
        in_specs=[pl.BlockSpec((NTILES, _FCOLS), lambda i: (0, i)),
                  pl.BlockSpec((1, _FCOLS), lambda i: (0, i))],
        out_specs=pl.BlockSpec((1, _FCOLS), lambda i: (0, i)),
        out_shape=jax.ShapeDtypeStruct((1, AWORDS), jnp.float32),
    )(parts, xf)


def kernel(x, edges, phases):
    E = phases.shape[0]
    pad = PAD_ROWS * COLS - E
    u = edges[:, 0].astype(jnp.int32)
    v = edges[:, 1].astype(jnp.int32)
    u2 = jnp.concatenate([u, jnp.full((pad,), N_NODES, jnp.int32)]).reshape(
        PAD_ROWS, COLS)
    v2 = jnp.concatenate([v, jnp.full((pad,), N_NODES, jnp.int32)]).reshape(
        PAD_ROWS, COLS)
    p2 = jnp.concatenate([phases, jnp.zeros((pad,), jnp.float32)]).reshape(
        PAD_ROWS, COLS)
    cs = _sincos_pack(p2)
    xb0 = lax.bitcast_convert_type(x[:, 0].astype(jnp.bfloat16),
                                   jnp.uint16).astype(jnp.int32)
    xb1 = lax.bitcast_convert_type(x[:, 1].astype(jnp.bfloat16),
                                   jnp.uint16).astype(jnp.int32)
    xp = jnp.concatenate([(xb1 << 16) | xb0,
                          jnp.zeros((NPAD - N_NODES,), jnp.int32)])
    va, vb, vc, vd = _rotate_call(xp, u2, v2, cs)
    zeros = jnp.zeros((NPAD,), jnp.float32)
    parts = _accum_call(u2, v2, va, vb, vc, vd, zeros)
    npad0 = jnp.zeros((NPAD - N_NODES,), jnp.float32)
    xf = jnp.concatenate([x[:, 0], npad0, x[:, 1], npad0])
    total = _finish(parts.reshape(NTILES, AWORDS), xf.reshape(1, AWORDS))
    return total.reshape(2, NPAD)[:, :N_NODES].T


# speedup vs baseline: 100.4795x; 1.0455x over previous
import jax
import jax.numpy as jnp
from jax import lax
from jax.experimental import pallas as pl
from jax.experimental.pallas import tpu as pltpu
from jax.experimental.pallas import tpu_sc as plsc

N_NODES = 100000
NPAD = 100096
AWORDS = 2 * NPAD
COLS = 128
PAD_ROWS = 50048
NCORES = 2
NSUB = 16
NTILES = NCORES * NSUB
RPT = PAD_ROWS // NTILES
K = 23
NCHUNK = RPT // K
LANES = 16

_CP = pltpu.CompilerParams(needs_layout_passes=False, use_tc_tiling_on_sc=False)
_MESH = plsc.VectorSubcoreMesh(core_axis_name="c", subcore_axis_name="s",
                               num_cores=NCORES, num_subcores=NSUB)


def _sincos_body(p_ref, cs_ref):
    p = p_ref[...]
    kf = jnp.round(p * 0.6366197723675814)
    ki = kf.astype(jnp.int32)
    r = p - kf * 1.5707963705062866
    r = r - kf * (-4.371139000186241e-08)
    r2 = r * r
    sp = r * (1.0 + r2 * (-1.0 / 6.0 + r2 * (1.0 / 120.0)))
    cp = 1.0 + r2 * (-0.5 + r2 * (1.0 / 24.0 + r2 * (-1.0 / 720.0)))
    b0 = (ki & 1) == 1
    b1 = (ki & 2) == 2
    s = jnp.where(b0, cp, sp)
    c = jnp.where(b0, sp, cp)
    s = jnp.where(b1, -s, s)
    c = jnp.where(b1 != b0, -c, c)
    c = c.astype(jnp.bfloat16)
    s = s.astype(jnp.bfloat16)
    cb = lax.bitcast_convert_type(c, jnp.uint16).astype(jnp.int32)
    sb = lax.bitcast_convert_type(s, jnp.uint16).astype(jnp.int32)
    cs_ref[...] = (sb << 16) | cb


def _sincos_pack(p2):
    return pl.pallas_call(
        _sincos_body,
        grid=(8,),
        in_specs=[pl.BlockSpec((PAD_ROWS // 8, COLS), lambda i: (i, 0))],
        out_specs=pl.BlockSpec((PAD_ROWS // 8, COLS), lambda i: (i, 0)),
        out_shape=jax.ShapeDtypeStruct((PAD_ROWS, COLS), jnp.int32),
    )(p2)


def _unpack_pair(w):
    a = plsc.bitcast(lax.shift_left(w, 16), jnp.float32)
    b = plsc.bitcast(lax.bitwise_and(w, jnp.int32(-65536)), jnp.float32)
    return a, b


def _rotate_body(xp_hbm, u_hbm, v_hbm, cs_hbm,
                 va_hbm, vb_hbm, vc_hbm, vd_hbm,
                 xp_v, u0, v0, cs0, u1, v1, cs1,
                 va_b, vb_b, vc_b, vd_b, si0, si1, so):
    c = lax.axis_index("c")
    s = lax.axis_index("s")
    wid = c * NSUB + s
    pltpu.sync_copy(xp_hbm, xp_v)
    rbeg = wid * RPT
    ins = ((u0, v0, cs0, si0), (u1, v1, cs1, si1))
    srcs = (u_hbm, v_hbm, cs_hbm)
    outs_hbm = (va_hbm, vb_hbm, vc_hbm, vd_hbm)
    outs_b = (va_b, vb_b, vc_b, vd_b)

    def fire(i, b):
        sl_r = pl.ds(rbeg + i * K, K)
        bu, bv, bc, sem = ins[b]
        pltpu.async_copy(u_hbm.at[sl_r], bu, sem)
        pltpu.async_copy(v_hbm.at[sl_r], bv, sem)
        pltpu.async_copy(cs_hbm.at[sl_r], bc, sem)

    def drain_in(b):
        bu, bv, bc, sem = ins[b]
        for src, dst in zip(srcs, (bu, bv, bc)):
            pltpu.make_async_copy(src.at[pl.ds(0, K)], dst, sem).wait()

    def drain_out():
        for src, dst in zip(outs_b, outs_hbm):
            pltpu.make_async_copy(src, dst.at[pl.ds(0, K)], so).wait()

    fire(0, 0)

    def group(g, carry):
        for b in range(2):
            i = g * 2 + b
            drain_in(b)

            @pl.when(i + 1 < NCHUNK)
            def _():
                fire(i + 1, 1 - b)

            @pl.when(i >= 1)
            def _():
                drain_out()

            bu, bv, bc, _ = ins[b]
            for k in range(K):
                for m in range(COLS // LANES):
                    sl = pl.ds(LANES * m, LANES)
                    u16 = bu[k, sl]
                    v16 = bv[k, sl]
                    cc, ss = _unpack_pair(bc[k, sl])
                    xu0, xu1 = _unpack_pair(plsc.load_gather(xp_v, [u16]))
                    xv0, xv1 = _unpack_pair(plsc.load_gather(xp_v, [v16]))
                    va_b[k, sl] = xu0 * cc - xu1 * ss
                    vb_b[k, sl] = xu0 * ss + xu1 * cc
                    vc_b[k, sl] = xv0 * cc + xv1 * ss
                    vd_b[k, sl] = xv1 * cc - xv0 * ss
            sl_r = pl.ds(rbeg + i * K, K)
            for src, dst in zip(outs_b, outs_hbm):
                pltpu.async_copy(src, dst.at[sl_r], so)
        return carry

    lax.fori_loop(0, NCHUNK // 2, group, 0)
    drain_out()


_rotate_call = pl.kernel(
    _rotate_body,
    out_type=[jax.ShapeDtypeStruct((PAD_ROWS, COLS), jnp.float32)] * 4,
    mesh=_MESH,
    compiler_params=_CP,
    scratch_types=[
        pltpu.VMEM((NPAD,), jnp.int32),
        pltpu.VMEM((K, COLS), jnp.int32),
        pltpu.VMEM((K, COLS), jnp.int32),
        pltpu.VMEM((K, COLS), jnp.int32),
        pltpu.VMEM((K, COLS), jnp.int32),
        pltpu.VMEM((K, COLS), jnp.int32),
        pltpu.VMEM((K, COLS), jnp.int32),
        pltpu.VMEM((K, COLS), jnp.float32),
        pltpu.VMEM((K, COLS), jnp.float32),
        pltpu.VMEM((K, COLS), jnp.float32),
        pltpu.VMEM((K, COLS), jnp.float32),
        pltpu.SemaphoreType.DMA,
        pltpu.SemaphoreType.DMA,
        pltpu.SemaphoreType.DMA,
    ],
)


def _accum_body(u_hbm, v_hbm, va_hbm, vb_hbm, vc_hbm, vd_hbm,
                out_hbm, u0, v0, a0, c0, u1, v1, a1, c1, acc, s0, s1):
    c = lax.axis_index("c")
    s = lax.axis_index("s")
    wid = c * NSUB + s
    rbeg = wid * RPT
    ins = ((u0, v0, a0, c0, s0), (u1, v1, a1, c1, s1))
    z16 = jnp.zeros((LANES,), jnp.float32)

    def zfill(i, carry):
        base = i * 256
        for j in range(256 // LANES):
            acc[pl.ds(base + j * LANES, LANES)] = z16
        return carry

    for p in range(2):
        A_hbm = va_hbm if p == 0 else vb_hbm
        C_hbm = vc_hbm if p == 0 else vd_hbm
        srcs = (u_hbm, v_hbm, A_hbm, C_hbm)
        lax.fori_loop(0, NPAD // 256, zfill, 0)

        def fire(i, b):
            sl_r = pl.ds(rbeg + i * K, K)
            bufs = ins[b]
            for src, dst in zip(srcs, bufs[:4]):
                pltpu.async_copy(src.at[sl_r], dst, bufs[4])

        def drain(b):
            bufs = ins[b]
            for src, dst in zip(srcs, bufs[:4]):
                pltpu.make_async_copy(src.at[pl.ds(0, K)], dst, bufs[4]).wait()

        fire(0, 0)

        def group(g, carry):
            for b in range(2):
                i = g * 2 + b
                drain(b)

                @pl.when(i + 1 < NCHUNK)
                def _():
                    fire(i + 1, 1 - b)

                bu, bv, ba, bcv, _ = ins[b]
                for k in range(K):
                    for m in range(COLS // LANES):
                        sl = pl.ds(LANES * m, LANES)
                        plsc.addupdate_scatter(acc, [bv[k, sl]], ba[k, sl])
                        plsc.addupdate_scatter(acc, [bu[k, sl]], bcv[k, sl])
            return carry

        lax.fori_loop(0, NCHUNK // 2, group, 0)
        pltpu.sync_copy(acc, out_hbm.at[pl.ds(wid * AWORDS + p * NPAD, NPAD)])


_accum_call = pl.kernel(
    _accum_body,
    out_type=jax.ShapeDtypeStruct((NTILES * AWORDS,), jnp.float32),
    mesh=_MESH,
    compiler_params=_CP,
    scratch_types=[
        pltpu.VMEM((K, COLS), jnp.int32),
        pltpu.VMEM((K, COLS), jnp.int32),
        pltpu.VMEM((K, COLS), jnp.float32),
        pltpu.VMEM((K, COLS), jnp.float32),
        pltpu.VMEM((K, COLS), jnp.int32),
        pltpu.VMEM((K, COLS), jnp.int32),
        pltpu.VMEM((K, COLS), jnp.float32),
        pltpu.VMEM((K, COLS), jnp.float32),
        pltpu.VMEM((NPAD,), jnp.float32),
        pltpu.SemaphoreType.DMA,
        pltpu.SemaphoreType.DMA,
    ],
)


_FCOLS = 5888


def _finish_body(parts_ref, x_ref, o_ref):
    o_ref[...] = jnp.sum(parts_ref[...], axis=0, keepdims=True) + x_ref[...]


def _finish(parts, xf):
    return pl.pallas_call(
        _finish_body,
        grid=(AWORDS // _FCOLS,),
        in_specs=[pl.BlockSpec((NTILES, _FCOLS), lambda i: (0, i)),
                  pl.BlockSpec((1, _FCOLS), lambda i: (0, i))],
        out_specs=pl.BlockSpec((1, _FCOLS), lambda i: (0, i)),
        out_shape=jax.ShapeDtypeStruct((1, AWORDS), jnp.float32),
    )(parts, xf)


def kernel(x, edges, phases):
    E = phases.shape[0]
    pad = PAD_ROWS * COLS - E
    u = edges[:, 0].astype(jnp.int32)
    v = edges[:, 1].astype(jnp.int32)
    u2 = jnp.concatenate([u, jnp.full((pad,), N_NODES, jnp.int32)]).reshape(
        PAD_ROWS, COLS)
    v2 = jnp.concatenate([v, jnp.full((pad,), N_NODES, jnp.int32)]).reshape(
        PAD_ROWS, COLS)
    p2 = jnp.concatenate([phases, jnp.zeros((pad,), jnp.float32)]).reshape(
        PAD_ROWS, COLS)
    cs = _sincos_pack(p2)
    xb0 = lax.bitcast_convert_type(x[:, 0].astype(jnp.bfloat16),
                                   jnp.uint16).astype(jnp.int32)
    xb1 = lax.bitcast_convert_type(x[:, 1].astype(jnp.bfloat16),
                                   jnp.uint16).astype(jnp.int32)
    xp = jnp.concatenate([(xb1 << 16) | xb0,
                          jnp.zeros((NPAD - N_NODES,), jnp.int32)])
    va, vb, vc, vd = _rotate_call(xp, u2, v2, cs)
    parts = _accum_call(u2, v2, va, vb, vc, vd)
    npad0 = jnp.zeros((NPAD - N_NODES,), jnp.float32)
    xf = jnp.concatenate([x[:, 0], npad0, x[:, 1], npad0])
    total = _finish(parts.reshape(NTILES, AWORDS), xf.reshape(1, AWORDS))
    return total.reshape(2, NPAD)[:, :N_NODES].T


# speedup vs baseline: 135.8912x; 1.3524x over previous
import jax
import jax.numpy as jnp
from jax import lax
from jax.experimental import pallas as pl
from jax.experimental.pallas import tpu as pltpu
from jax.experimental.pallas import tpu_sc as plsc

N_NODES = 100000
NPAD = 100096
AWORDS = 2 * NPAD
COLS = 128
PAD_ROWS = 50048
NCORES = 2
NSUB = 16
NTILES = NCORES * NSUB
RPT = PAD_ROWS // NTILES
K = 23
NCHUNK = RPT // K
LANES = 16

_CP = pltpu.CompilerParams(needs_layout_passes=False, use_tc_tiling_on_sc=False)
_MESH = plsc.VectorSubcoreMesh(core_axis_name="c", subcore_axis_name="s",
                               num_cores=NCORES, num_subcores=NSUB)


def _sincos_body(p_ref, cs_ref):
    p = p_ref[...]
    kf = jnp.round(p * 0.6366197723675814)
    ki = kf.astype(jnp.int32)
    r = p - kf * 1.5707963705062866
    r = r - kf * (-4.371139000186241e-08)
    r2 = r * r
    sp = r * (1.0 + r2 * (-1.0 / 6.0 + r2 * (1.0 / 120.0)))
    cp = 1.0 + r2 * (-0.5 + r2 * (1.0 / 24.0 + r2 * (-1.0 / 720.0)))
    b0 = (ki & 1) == 1
    b1 = (ki & 2) == 2
    s = jnp.where(b0, cp, sp)
    c = jnp.where(b0, sp, cp)
    s = jnp.where(b1, -s, s)
    c = jnp.where(b1 != b0, -c, c)
    c = c.astype(jnp.bfloat16)
    s = s.astype(jnp.bfloat16)
    cb = lax.bitcast_convert_type(c, jnp.uint16).astype(jnp.int32)
    sb = lax.bitcast_convert_type(s, jnp.uint16).astype(jnp.int32)
    cs_ref[...] = (sb << 16) | cb


def _sincos_pack(p2):
    return pl.pallas_call(
        _sincos_body,
        grid=(8,),
        in_specs=[pl.BlockSpec((PAD_ROWS // 8, COLS), lambda i: (i, 0))],
        out_specs=pl.BlockSpec((PAD_ROWS // 8, COLS), lambda i: (i, 0)),
        out_shape=jax.ShapeDtypeStruct((PAD_ROWS, COLS), jnp.int32),
    )(p2)


def _unpack_pair(w):
    a = plsc.bitcast(lax.shift_left(w, 16), jnp.float32)
    b = plsc.bitcast(lax.bitwise_and(w, jnp.int32(-65536)), jnp.float32)
    return a, b


def _rotate_body(xp_hbm, u_hbm, v_hbm, cs_hbm,
                 va_hbm, vb_hbm, vc_hbm, vd_hbm,
                 xp_v, u0, v0, cs0, u1, v1, cs1,
                 va_b, vb_b, vc_b, vd_b, si0, si1, so):
    c = lax.axis_index("c")
    s = lax.axis_index("s")
    wid = c * NSUB + s
    pltpu.sync_copy(xp_hbm, xp_v)
    rbeg = wid * RPT
    ins = ((u0, v0, cs0, si0), (u1, v1, cs1, si1))
    srcs = (u_hbm, v_hbm, cs_hbm)
    outs_hbm = (va_hbm, vb_hbm, vc_hbm, vd_hbm)
    outs_b = (va_b, vb_b, vc_b, vd_b)

    def fire(i, b):
        sl_r = pl.ds(rbeg + i * K, K)
        bu, bv, bc, sem = ins[b]
        pltpu.async_copy(u_hbm.at[sl_r], bu, sem)
        pltpu.async_copy(v_hbm.at[sl_r], bv, sem)
        pltpu.async_copy(cs_hbm.at[sl_r], bc, sem)

    def drain_in(b):
        bu, bv, bc, sem = ins[b]
        for src, dst in zip(srcs, (bu, bv, bc)):
            pltpu.make_async_copy(src.at[pl.ds(0, K)], dst, sem).wait()

    def drain_out():
        for src, dst in zip(outs_b, outs_hbm):
            pltpu.make_async_copy(src, dst.at[pl.ds(0, K)], so).wait()

    fire(0, 0)

    def group(g, carry):
        for b in range(2):
            i = g * 2 + b
            drain_in(b)

            @pl.when(i + 1 < NCHUNK)
            def _():
                fire(i + 1, 1 - b)

            @pl.when(i >= 1)
            def _():
                drain_out()

            bu, bv, bc, _ = ins[b]
            MB = COLS // LANES

            def loads(k, m):
                sl = pl.ds(LANES * m, LANES)
                return bu[k, sl], bv[k, sl], bc[k, sl]

            def gathers(l):
                u16, v16, _ = l
                return (plsc.load_gather(xp_v, [u16]),
                        plsc.load_gather(xp_v, [v16]))

            flat = [(k, m) for k in range(K) for m in range(MB)]
            l0 = loads(0, 0)
            g0 = gathers(l0)
            l1 = loads(0, 1)
            for idx, (k, m) in enumerate(flat):
                l2 = loads(*flat[idx + 2]) if idx + 2 < len(flat) else None
                g1 = gathers(l1) if l1 is not None else None
                sl = pl.ds(LANES * m, LANES)
                cc, ss = _unpack_pair(l0[2])
                xu0, xu1 = _unpack_pair(g0[0])
                xv0, xv1 = _unpack_pair(g0[1])
                va_b[k, sl] = xu0 * cc - xu1 * ss
                vb_b[k, sl] = xu0 * ss + xu1 * cc
                vc_b[k, sl] = xv0 * cc + xv1 * ss
                vd_b[k, sl] = xv1 * cc - xv0 * ss
                l0, g0, l1 = l1, g1, l2
            sl_r = pl.ds(rbeg + i * K, K)
            for src, dst in zip(outs_b, outs_hbm):
                pltpu.async_copy(src, dst.at[sl_r], so)
        return carry

    lax.fori_loop(0, NCHUNK // 2, group, 0)
    drain_out()


_rotate_call = pl.kernel(
    _rotate_body,
    out_type=[jax.ShapeDtypeStruct((PAD_ROWS, COLS), jnp.float32)] * 4,
    mesh=_MESH,
    compiler_params=_CP,
    scratch_types=[
        pltpu.VMEM((NPAD,), jnp.int32),
        pltpu.VMEM((K, COLS), jnp.int32),
        pltpu.VMEM((K, COLS), jnp.int32),
        pltpu.VMEM((K, COLS), jnp.int32),
        pltpu.VMEM((K, COLS), jnp.int32),
        pltpu.VMEM((K, COLS), jnp.int32),
        pltpu.VMEM((K, COLS), jnp.int32),
        pltpu.VMEM((K, COLS), jnp.float32),
        pltpu.VMEM((K, COLS), jnp.float32),
        pltpu.VMEM((K, COLS), jnp.float32),
        pltpu.VMEM((K, COLS), jnp.float32),
        pltpu.SemaphoreType.DMA,
        pltpu.SemaphoreType.DMA,
        pltpu.SemaphoreType.DMA,
    ],
)


def _accum_body(u_hbm, v_hbm, va_hbm, vb_hbm, vc_hbm, vd_hbm,
                out_hbm, u0, v0, a0, c0, u1, v1, a1, c1, acc, s0, s1):
    c = lax.axis_index("c")
    s = lax.axis_index("s")
    wid = c * NSUB + s
    rbeg = wid * RPT
    ins = ((u0, v0, a0, c0, s0), (u1, v1, a1, c1, s1))
    z16 = jnp.zeros((LANES,), jnp.float32)

    def zfill(i, carry):
        base = i * 256
        for j in range(256 // LANES):
            acc[pl.ds(base + j * LANES, LANES)] = z16
        return carry

    for p in range(2):
        A_hbm = va_hbm if p == 0 else vb_hbm
        C_hbm = vc_hbm if p == 0 else vd_hbm
        srcs = (u_hbm, v_hbm, A_hbm, C_hbm)
        lax.fori_loop(0, NPAD // 256, zfill, 0)

        def fire(i, b):
            sl_r = pl.ds(rbeg + i * K, K)
            bufs = ins[b]
            for src, dst in zip(srcs, bufs[:4]):
                pltpu.async_copy(src.at[sl_r], dst, bufs[4])

        def drain(b):
            bufs = ins[b]
            for src, dst in zip(srcs, bufs[:4]):
                pltpu.make_async_copy(src.at[pl.ds(0, K)], dst, bufs[4]).wait()

        fire(0, 0)

        def group(g, carry):
            for b in range(2):
                i = g * 2 + b
                drain(b)

                @pl.when(i + 1 < NCHUNK)
                def _():
                    fire(i + 1, 1 - b)

                bu, bv, ba, bcv, _ = ins[b]
                MB = COLS // LANES

                def loads(k, m):
                    sl = pl.ds(LANES * m, LANES)
                    return bv[k, sl], ba[k, sl], bu[k, sl], bcv[k, sl]

                cur = loads(0, 0)
                for k in range(K):
                    for m in range(MB):
                        nk, nm = (k, m + 1) if m + 1 < MB else (k + 1, 0)
                        nxt = loads(nk, nm) if nk < K else None
                        vv, aa, uu, cc2 = cur
                        plsc.addupdate_scatter(acc, [vv], aa)
                        plsc.addupdate_scatter(acc, [uu], cc2)
                        cur = nxt
            return carry

        lax.fori_loop(0, NCHUNK // 2, group, 0)
        pltpu.sync_copy(acc, out_hbm.at[pl.ds(wid * AWORDS + p * NPAD, NPAD)])


_accum_call = pl.kernel(
    _accum_body,
    out_type=jax.ShapeDtypeStruct((NTILES * AWORDS,), jnp.float32),
    mesh=_MESH,
    compiler_params=_CP,
    scratch_types=[
        pltpu.VMEM((K, COLS), jnp.int32),
        pltpu.VMEM((K, COLS), jnp.int32),
        pltpu.VMEM((K, COLS), jnp.float32),
        pltpu.VMEM((K, COLS), jnp.float32),
        pltpu.VMEM((K, COLS), jnp.int32),
        pltpu.VMEM((K, COLS), jnp.int32),
        pltpu.VMEM((K, COLS), jnp.float32),
        pltpu.VMEM((K, COLS), jnp.float32),
        pltpu.VMEM((NPAD,), jnp.float32),
        pltpu.SemaphoreType.DMA,
        pltpu.SemaphoreType.DMA,
    ],
)


_FCOLS = 5888


def _finish_body(parts_ref, x_ref, o_ref):
    o_ref[...] = jnp.sum(parts_ref[...], axis=0, keepdims=True) + x_ref[...]


def _finish(parts, xf):
    return pl.pallas_call(
        _finish_body,
        grid=(AWORDS // _FCOLS,),
        in_specs=[pl.BlockSpec((NTILES, _FCOLS), lambda i: (0, i)),
                  pl.BlockSpec((1, _FCOLS), lambda i: (0, i))],
        out_specs=pl.BlockSpec((1, _FCOLS), lambda i: (0, i)),
        out_shape=jax.ShapeDtypeStruct((1, AWORDS), jnp.float32),
    )(parts, xf)


def kernel(x, edges, phases):
    E = phases.shape[0]
    pad = PAD_ROWS * COLS - E
    u = edges[:, 0].astype(jnp.int32)
    v = edges[:, 1].astype(jnp.int32)
    u2 = jnp.concatenate([u, jnp.full((pad,), N_NODES, jnp.int32)]).reshape(
        PAD_ROWS, COLS)
    v2 = jnp.concatenate([v, jnp.full((pad,), N_NODES, jnp.int32)]).reshape(
        PAD_ROWS, COLS)
    p2 = jnp.concatenate([phases, jnp.zeros((pad,), jnp.float32)]).reshape(
        PAD_ROWS, COLS)
    cs = _sincos_pack(p2)
    xb0 = lax.bitcast_convert_type(x[:, 0].astype(jnp.bfloat16),
                                   jnp.uint16).astype(jnp.int32)
    xb1 = lax.bitcast_convert_type(x[:, 1].astype(jnp.bfloat16),
                                   jnp.uint16).astype(jnp.int32)
    xp = jnp.concatenate([(xb1 << 16) | xb0,
                          jnp.zeros((NPAD - N_NODES,), jnp.int32)])
    va, vb, vc, vd = _rotate_call(xp, u2, v2, cs)
    parts = _accum_call(u2, v2, va, vb, vc, vd)
    npad0 = jnp.zeros((NPAD - N_NODES,), jnp.float32)
    xf = jnp.concatenate([x[:, 0], npad0, x[:, 1], npad0])
    total = _finish(parts.reshape(NTILES, AWORDS), xf.reshape(1, AWORDS))
    return total.reshape(2, NPAD)[:, :N_NODES].T
